# baseline (speedup 1.0000x reference)
import functools

import jax
import jax.numpy as jnp
from jax import lax
from jax.experimental import pallas as pl
from jax.experimental.pallas import tpu as pltpu
from jax.experimental.pallas import tpu_sc as plsc


_BLKC = 32768


def _pack_body(embT_ref, w_ref, b_ref, out_ref):
    h = jnp.maximum(embT_ref[...], 0.0)
    f = (
        lax.dot_general(
            h, w_ref[...], (((0,), (1,)), ((), ())),
            preferred_element_type=jnp.float32,
        )
        + b_ref[...]
    )
    out_ref[...] = jnp.concatenate([f, jnp.zeros_like(f)], axis=1)


def _transform_table(embT, W, b2d):
    hidden, n_rows = embT.shape
    out_dim = W.shape[0]
    grid = (pl.cdiv(n_rows, _BLKC),)
    return pl.pallas_call(
        _pack_body,
        grid=grid,
        in_specs=[
            pl.BlockSpec((hidden, _BLKC), lambda i: (0, i)),
            pl.BlockSpec((out_dim, hidden), lambda i: (0, 0)),
            pl.BlockSpec((1, out_dim), lambda i: (0, 0)),
        ],
        out_specs=pl.BlockSpec((_BLKC, 2 * out_dim), lambda i: (i, 0)),
        out_shape=jax.ShapeDtypeStruct((n_rows, 2 * out_dim), jnp.float32),
        compiler_params=pltpu.CompilerParams(
            dimension_semantics=("arbitrary",),
        ),
    )(embT, W, b2d)


_NC = 2
_NS = 16
_NW = _NC * _NS
_CH = 128
_NBUF = 6


def _make_gather(n_idx, width):
    n_chunks = n_idx // (_NW * _CH)
    rows_per_w = n_chunks * _CH
    mesh = plsc.VectorSubcoreMesh(core_axis_name="c", subcore_axis_name="s")

    @functools.partial(
        pl.kernel,
        mesh=mesh,
        out_type=jax.ShapeDtypeStruct((n_idx, width), jnp.float32),
        scratch_types=[
            pltpu.VMEM((n_chunks, _CH), jnp.int32),
            pltpu.VMEM((_NBUF, _CH, width), jnp.float32),
            pltpu.SemaphoreType.DMA,
            pltpu.SemaphoreType.DMA,
        ],
    )
    def gather_k(table_hbm, idx_hbm, out_hbm, idx_v, rows_v, gsem, osem):
        wid = lax.axis_index("s") * _NC + lax.axis_index("c")
        base = wid * rows_per_w
        pltpu.sync_copy(idx_hbm.at[wid], idx_v)

        ahead = _NBUF // 2

        def issue(j):
            return pltpu.async_copy(
                table_hbm.at[idx_v.at[j]], rows_v.at[lax.rem(j, _NBUF)], gsem
            )

        def write(j):
            return pltpu.async_copy(
                rows_v.at[lax.rem(j, _NBUF)],
                out_hbm.at[pl.ds(base + j * _CH, _CH)],
                osem,
            )

        for j in range(ahead):
            issue(j)

        def step(j, carry):
            buf = lax.rem(j, _NBUF)
            pltpu.make_async_copy(
                table_hbm.at[idx_v.at[j]], rows_v.at[buf], gsem
            ).wait()
            write(j)

            @pl.when(j >= ahead)
            def _():
                jb = j - ahead
                pltpu.make_async_copy(
                    rows_v.at[lax.rem(jb, _NBUF)],
                    out_hbm.at[pl.ds(base + jb * _CH, _CH)],
                    osem,
                ).wait()

            @pl.when(j + ahead < n_chunks)
            def _():
                issue(j + ahead)

            return carry

        lax.fori_loop(0, n_chunks, step, 0, unroll=False)
        for t in range(ahead):
            jb = n_chunks - ahead + t
            pltpu.make_async_copy(
                rows_v.at[jb % _NBUF],
                out_hbm.at[pl.ds(base + jb * _CH, _CH)],
                osem,
            ).wait()

    return gather_k


_BLKB = 4096
_BLKL = 2


def _transpose_body(g_ref, eye_ref, out_ref):
    for l in range(_BLKL):
        out_ref[l] = lax.dot_general(
            eye_ref[...], g_ref[l], (((1,), (1,)), ((), ())),
            preferred_element_type=jnp.float32,
        )


def _transpose_out(g3, out_dim):
    L, B, width = g3.shape
    grid = (L // _BLKL, B // _BLKB)
    eye = jnp.concatenate(
        [jnp.eye(out_dim, dtype=jnp.float32),
         jnp.zeros((out_dim, width - out_dim), jnp.float32)], axis=1)
    return pl.pallas_call(
        _transpose_body,
        grid=grid,
        in_specs=[
            pl.BlockSpec((_BLKL, _BLKB, width), lambda l, i: (l, i, 0)),
            pl.BlockSpec((out_dim, width), lambda l, i: (0, 0)),
        ],
        out_specs=pl.BlockSpec((_BLKL, out_dim, _BLKB), lambda l, i: (l, 0, i)),
        out_shape=jax.ShapeDtypeStruct((L, out_dim, B), jnp.float32),
        compiler_params=pltpu.CompilerParams(
            dimension_semantics=("arbitrary", "arbitrary"),
        ),
    )(g3, eye)


def kernel(X, emb, W, b):
    B, L = X.shape
    n_rows, hidden = emb.shape
    out_dim = W.shape[0]
    n_idx = B * L
    n_chunks = n_idx // (_NW * _CH)
    assert n_idx == _NW * n_chunks * _CH

    table = _transform_table(emb.T, W, b.reshape(1, out_dim))
    idx = X.T.astype(jnp.int32).reshape(_NW, n_chunks, _CH)
    g = _make_gather(n_idx, 2 * out_dim)(table, idx)
    out = _transpose_out(g.reshape(L, B, 2 * out_dim), out_dim)
    return jnp.transpose(out, (2, 0, 1))


# speedup vs baseline: 1.0175x; 1.0175x over previous
import functools

import jax
import jax.numpy as jnp
from jax import lax
from jax.experimental import pallas as pl
from jax.experimental.pallas import tpu as pltpu
from jax.experimental.pallas import tpu_sc as plsc


_BLKC = 32768


def _pack_body(embT_ref, w_ref, b_ref, out_ref):
    h = jnp.maximum(embT_ref[...], 0.0)
    f = (
        lax.dot_general(
            h, w_ref[...], (((0,), (1,)), ((), ())),
            preferred_element_type=jnp.float32,
        )
        + b_ref[...]
    )
    out_ref[...] = jnp.concatenate([f, jnp.zeros_like(f)], axis=1)


def _transform_table(embT, W, b2d):
    hidden, n_rows = embT.shape
    out_dim = W.shape[0]
    grid = (pl.cdiv(n_rows, _BLKC),)
    return pl.pallas_call(
        _pack_body,
        grid=grid,
        in_specs=[
            pl.BlockSpec((hidden, _BLKC), lambda i: (0, i)),
            pl.BlockSpec((out_dim, hidden), lambda i: (0, 0)),
            pl.BlockSpec((1, out_dim), lambda i: (0, 0)),
        ],
        out_specs=pl.BlockSpec((_BLKC, 2 * out_dim), lambda i: (i, 0)),
        out_shape=jax.ShapeDtypeStruct((n_rows, 2 * out_dim), jnp.float32),
        compiler_params=pltpu.CompilerParams(
            dimension_semantics=("arbitrary",),
        ),
    )(embT, W, b2d)


_NC = 2
_NS = 16
_NW = _NC * _NS
_CH = 128
_NBUF = 6


def _make_gather(n_idx, width):
    n_chunks = n_idx // (_NW * _CH)
    rows_per_w = n_chunks * _CH
    mesh = plsc.VectorSubcoreMesh(core_axis_name="c", subcore_axis_name="s")

    @functools.partial(
        pl.kernel,
        mesh=mesh,
        out_type=jax.ShapeDtypeStruct((n_idx, width), jnp.float32),
        scratch_types=[
            pltpu.VMEM((n_chunks, _CH), jnp.int32),
            pltpu.VMEM((_NBUF, _CH, width), jnp.float32),
            pltpu.SemaphoreType.DMA,
            pltpu.SemaphoreType.DMA,
        ],
    )
    def gather_k(table_hbm, idx_hbm, out_hbm, idx_v, rows_v, gsem, osem):
        wid = lax.axis_index("s") * _NC + lax.axis_index("c")
        base = wid * rows_per_w
        pltpu.sync_copy(idx_hbm.at[wid], idx_v)

        ahead = _NBUF // 2

        def issue(j):
            return pltpu.async_copy(
                table_hbm.at[idx_v.at[j]], rows_v.at[lax.rem(j, _NBUF)], gsem
            )

        def write(j):
            return pltpu.async_copy(
                rows_v.at[lax.rem(j, _NBUF)],
                out_hbm.at[pl.ds(base + j * _CH, _CH)],
                osem,
            )

        for j in range(ahead):
            issue(j)

        def step(j, carry):
            buf = lax.rem(j, _NBUF)
            pltpu.make_async_copy(
                table_hbm.at[idx_v.at[j]], rows_v.at[buf], gsem
            ).wait()
            write(j)

            @pl.when(j >= ahead)
            def _():
                jb = j - ahead
                pltpu.make_async_copy(
                    rows_v.at[lax.rem(jb, _NBUF)],
                    out_hbm.at[pl.ds(base + jb * _CH, _CH)],
                    osem,
                ).wait()

            @pl.when(j + ahead < n_chunks)
            def _():
                issue(j + ahead)

            return carry

        lax.fori_loop(0, n_chunks, step, 0, unroll=False)
        for t in range(ahead):
            jb = n_chunks - ahead + t
            pltpu.make_async_copy(
                rows_v.at[jb % _NBUF],
                out_hbm.at[pl.ds(base + jb * _CH, _CH)],
                osem,
            ).wait()

    return gather_k


_BLKB = 8192
_BLKL = 2


def _transpose_body(g_ref, eye_ref, out_ref):
    for l in range(_BLKL):
        out_ref[l] = lax.dot_general(
            eye_ref[...], g_ref[l], (((1,), (1,)), ((), ())),
            preferred_element_type=jnp.float32,
        )


def _transpose_out(g3, out_dim):
    L, B, width = g3.shape
    grid = (L // _BLKL, B // _BLKB)
    eye = jnp.concatenate(
        [jnp.eye(out_dim, dtype=jnp.float32),
         jnp.zeros((out_dim, width - out_dim), jnp.float32)], axis=1)
    return pl.pallas_call(
        _transpose_body,
        grid=grid,
        in_specs=[
            pl.BlockSpec((_BLKL, _BLKB, width), lambda l, i: (l, i, 0)),
            pl.BlockSpec((out_dim, width), lambda l, i: (0, 0)),
        ],
        out_specs=pl.BlockSpec((_BLKL, out_dim, _BLKB), lambda l, i: (l, 0, i)),
        out_shape=jax.ShapeDtypeStruct((L, out_dim, B), jnp.float32),
        compiler_params=pltpu.CompilerParams(
            dimension_semantics=("arbitrary", "arbitrary"),
        ),
    )(g3, eye)


def kernel(X, emb, W, b):
    B, L = X.shape
    n_rows, hidden = emb.shape
    out_dim = W.shape[0]
    n_idx = B * L
    n_chunks = n_idx // (_NW * _CH)
    assert n_idx == _NW * n_chunks * _CH

    table = _transform_table(emb.T, W, b.reshape(1, out_dim))
    idx = X.T.astype(jnp.int32).reshape(_NW, n_chunks, _CH)
    g = _make_gather(n_idx, 2 * out_dim)(table, idx)
    out = _transpose_out(g.reshape(L, B, 2 * out_dim), out_dim)
    return jnp.transpose(out, (2, 0, 1))


# speedup vs baseline: 1.0197x; 1.0021x over previous
import functools

import jax
import jax.numpy as jnp
from jax import lax
from jax.experimental import pallas as pl
from jax.experimental.pallas import tpu as pltpu
from jax.experimental.pallas import tpu_sc as plsc


_BLKC = 32768


def _pack_body(embT_ref, w_ref, b_ref, out_ref):
    h = jnp.maximum(embT_ref[...], 0.0)
    f = (
        lax.dot_general(
            h, w_ref[...], (((0,), (1,)), ((), ())),
            preferred_element_type=jnp.float32,
        )
        + b_ref[...]
    )
    out_ref[...] = jnp.concatenate([f, jnp.zeros_like(f)], axis=1)


def _transform_table(embT, W, b2d):
    hidden, n_rows = embT.shape
    out_dim = W.shape[0]
    grid = (pl.cdiv(n_rows, _BLKC),)
    return pl.pallas_call(
        _pack_body,
        grid=grid,
        in_specs=[
            pl.BlockSpec((hidden, _BLKC), lambda i: (0, i)),
            pl.BlockSpec((out_dim, hidden), lambda i: (0, 0)),
            pl.BlockSpec((1, out_dim), lambda i: (0, 0)),
        ],
        out_specs=pl.BlockSpec((_BLKC, 2 * out_dim), lambda i: (i, 0)),
        out_shape=jax.ShapeDtypeStruct((n_rows, 2 * out_dim), jnp.float32),
        compiler_params=pltpu.CompilerParams(
            dimension_semantics=("arbitrary",),
        ),
    )(embT, W, b2d)


_NC = 2
_NS = 16
_NW = _NC * _NS
_CH = 128
_NBUF = 6


def _make_gather(n_idx, width):
    n_chunks = n_idx // (_NW * _CH)
    rows_per_w = n_chunks * _CH
    mesh = plsc.VectorSubcoreMesh(core_axis_name="c", subcore_axis_name="s")

    @functools.partial(
        pl.kernel,
        mesh=mesh,
        out_type=jax.ShapeDtypeStruct((n_idx, width), jnp.float32),
        scratch_types=[
            pltpu.VMEM((n_chunks, _CH), jnp.int32),
            pltpu.VMEM((_NBUF, _CH, width), jnp.float32),
            pltpu.SemaphoreType.DMA,
            pltpu.SemaphoreType.DMA,
        ],
    )
    def gather_k(table_hbm, idx_hbm, out_hbm, idx_v, rows_v, gsem, osem):
        wid = lax.axis_index("s") * _NC + lax.axis_index("c")
        base = wid * rows_per_w
        pltpu.sync_copy(idx_hbm.at[wid], idx_v)

        ahead = _NBUF // 2

        def issue(j):
            return pltpu.async_copy(
                table_hbm.at[idx_v.at[j]], rows_v.at[lax.rem(j, _NBUF)], gsem
            )

        def write(j):
            return pltpu.async_copy(
                rows_v.at[lax.rem(j, _NBUF)],
                out_hbm.at[pl.ds(base + j * _CH, _CH)],
                osem,
            )

        for j in range(ahead):
            issue(j)

        def step(j, carry):
            buf = lax.rem(j, _NBUF)
            pltpu.make_async_copy(
                table_hbm.at[idx_v.at[j]], rows_v.at[buf], gsem
            ).wait()
            write(j)

            @pl.when(j >= ahead)
            def _():
                jb = j - ahead
                pltpu.make_async_copy(
                    rows_v.at[lax.rem(jb, _NBUF)],
                    out_hbm.at[pl.ds(base + jb * _CH, _CH)],
                    osem,
                ).wait()

            @pl.when(j + ahead < n_chunks)
            def _():
                issue(j + ahead)

            return carry

        lax.fori_loop(0, n_chunks, step, 0, unroll=False)
        for t in range(ahead):
            jb = n_chunks - ahead + t
            pltpu.make_async_copy(
                rows_v.at[jb % _NBUF],
                out_hbm.at[pl.ds(base + jb * _CH, _CH)],
                osem,
            ).wait()

    return gather_k


_BLKB = 4096
_BLKL = 5


def _transpose_body(g_ref, eye_ref, out_ref):
    for l in range(_BLKL):
        out_ref[l] = lax.dot_general(
            eye_ref[...], g_ref[l], (((1,), (1,)), ((), ())),
            preferred_element_type=jnp.float32,
        )


def _transpose_out(g3, out_dim):
    L, B, width = g3.shape
    grid = (L // _BLKL, B // _BLKB)
    eye = jnp.concatenate(
        [jnp.eye(out_dim, dtype=jnp.float32),
         jnp.zeros((out_dim, width - out_dim), jnp.float32)], axis=1)
    return pl.pallas_call(
        _transpose_body,
        grid=grid,
        in_specs=[
            pl.BlockSpec((_BLKL, _BLKB, width), lambda l, i: (l, i, 0)),
            pl.BlockSpec((out_dim, width), lambda l, i: (0, 0)),
        ],
        out_specs=pl.BlockSpec((_BLKL, out_dim, _BLKB), lambda l, i: (l, 0, i)),
        out_shape=jax.ShapeDtypeStruct((L, out_dim, B), jnp.float32),
        compiler_params=pltpu.CompilerParams(
            dimension_semantics=("arbitrary", "arbitrary"),
        ),
    )(g3, eye)


def kernel(X, emb, W, b):
    B, L = X.shape
    n_rows, hidden = emb.shape
    out_dim = W.shape[0]
    n_idx = B * L
    n_chunks = n_idx // (_NW * _CH)
    assert n_idx == _NW * n_chunks * _CH

    table = _transform_table(emb.T, W, b.reshape(1, out_dim))
    idx = X.T.astype(jnp.int32).reshape(_NW, n_chunks, _CH)
    g = _make_gather(n_idx, 2 * out_dim)(table, idx)
    out = _transpose_out(g.reshape(L, B, 2 * out_dim), out_dim)
    return jnp.transpose(out, (2, 0, 1))
